# baseline (speedup 1.0000x reference)
import jax
import jax.numpy as jnp
from jax.experimental import pallas as pl
from jax.experimental.pallas import tpu as pltpu

_B, _S, _D_IN, _D_OUT, _E, _R = 4, 2048, 2048, 2048, 8, 8
_SCALING = 16.0 / 8.0
_BS = 512


def _fused_body(ids_ref, x_ref, w_hbm, bias_ref, a_ref, bb_ref, o_ref,
                wvm_ref, wm_ref, sem):
    bi = pl.program_id(0)
    si = pl.program_id(1)
    dn = (((1,), (0,)), ((), ()))

    @pl.when((bi == 0) & (si == 0))
    def _fetch_w():
        cp = pltpu.make_async_copy(w_hbm, wvm_ref, sem)
        cp.start()
        cp.wait()

    @pl.when(si == 0)
    def _merge():
        upd = jax.lax.dot_general(a_ref[0], bb_ref[0], dn,
                                  preferred_element_type=jnp.float32)
        wm_ref[...] = wvm_ref[...] + upd

    acc = jax.lax.dot_general(x_ref[0], wm_ref[...], dn,
                              preferred_element_type=jnp.float32)
    o_ref[0] = acc + bias_ref[...]


def kernel(hidden_states, adapter_ids, W, b, lora_a, lora_b):
    ids = adapter_ids.astype(jnp.int32)
    bias2 = b.reshape(1, _D_OUT)
    bb_scaled = lora_b * _SCALING
    grid_spec = pltpu.PrefetchScalarGridSpec(
        num_scalar_prefetch=1,
        grid=(_B, _S // _BS),
        in_specs=[
            pl.BlockSpec((1, _BS, _D_IN), lambda bi, si, ids: (bi, si, 0)),
            pl.BlockSpec(memory_space=pl.ANY),
            pl.BlockSpec((1, _D_OUT), lambda bi, si, ids: (0, 0)),
            pl.BlockSpec((1, _D_IN, _R), lambda bi, si, ids: (ids[bi], 0, 0)),
            pl.BlockSpec((1, _R, _D_OUT), lambda bi, si, ids: (ids[bi], 0, 0)),
        ],
        out_specs=pl.BlockSpec((1, _BS, _D_OUT), lambda bi, si, ids: (bi, si, 0)),
        scratch_shapes=[
            pltpu.VMEM((_D_IN, _D_OUT), jnp.float32),
            pltpu.VMEM((_D_IN, _D_OUT), jnp.float32),
            pltpu.SemaphoreType.DMA,
        ],
    )
    return pl.pallas_call(
        _fused_body,
        grid_spec=grid_spec,
        out_shape=jax.ShapeDtypeStruct((_B, _S, _D_OUT), jnp.float32),
    )(ids, hidden_states, W, bias2, lora_a, bb_scaled)
